# baseline (speedup 1.0000x reference)
import jax
import jax.numpy as jnp
from jax.experimental import pallas as pl
from jax.experimental.pallas import tpu as pltpu

B = 1024
M = 100000
KD = 16
VD = 16
M_BLK = 2000
NB = M // M_BLK
LOG2E = 1.4426950408889634


def _body(q_ref, k_ref, v_ref, o_ref, km2_ref, qext_ref, acc_ref):
    phase = pl.program_id(0)
    i = pl.program_id(1)

    @pl.when(phase == 0)
    def _scan_key_norms():
        k = k_ref[:, 0:KD]
        n2 = jnp.sum(k * k, axis=1, keepdims=True)
        bmax = jnp.max(n2, axis=0, keepdims=True)

        @pl.when(i == 0)
        def _():
            km2_ref[...] = bmax

        @pl.when(i > 0)
        def _():
            km2_ref[...] = jnp.maximum(km2_ref[...], bmax)

    @pl.when(phase == 1)
    def _readout():
        @pl.when(i == 0)
        def _():
            q = q_ref[...]
            qn = jnp.sum(q * q, axis=1, keepdims=True)
            u = jnp.sqrt(qn * km2_ref[...])
            qext_ref[:, 0:KD] = q * LOG2E
            qext_ref[:, KD : KD + 1] = -(u * LOG2E)
            acc_ref[...] = jnp.zeros_like(acc_ref)

        s = jax.lax.dot_general(
            qext_ref[...],
            k_ref[...],
            (((1,), (1,)), ((), ())),
            preferred_element_type=jnp.float32,
        )
        p = jnp.exp2(s)
        acc_ref[...] += jnp.dot(
            p, v_ref[...], preferred_element_type=jnp.float32
        )

        @pl.when(i == NB - 1)
        def _():
            o_ref[...] = acc_ref[:, 0:VD] / acc_ref[:, VD : VD + 1]


@jax.jit
def kernel(query, keys, values):
    ones = jnp.ones((M, 1), jnp.float32)
    k_ext = jnp.concatenate([keys, ones], axis=1)
    v_ext = jnp.concatenate([values, ones], axis=1)
    return pl.pallas_call(
        _body,
        grid=(2, NB),
        in_specs=[
            pl.BlockSpec((B, KD), lambda p, i: (0, 0)),
            pl.BlockSpec((M_BLK, KD + 1), lambda p, i: (i, 0)),
            pl.BlockSpec((M_BLK, VD + 1), lambda p, i: (p * i, 0)),
        ],
        out_specs=pl.BlockSpec((B, VD), lambda p, i: (0, 0)),
        out_shape=jax.ShapeDtypeStruct((B, VD), jnp.float32),
        scratch_shapes=[
            pltpu.VMEM((1, 1), jnp.float32),
            pltpu.VMEM((B, KD + 1), jnp.float32),
            pltpu.VMEM((B, VD + 1), jnp.float32),
        ],
    )(query, k_ext, v_ext)


# speedup vs baseline: 1.3451x; 1.3451x over previous
import jax
import jax.numpy as jnp
from jax.experimental import pallas as pl
from jax.experimental.pallas import tpu as pltpu

B = 1024
M = 100000
KD = 16
VD = 16
M_BLK = 2000
NB = M // M_BLK
N0 = 10
M0_BLK = M // N0
LOG2E = 1.4426950408889634


def _body(q_ref, ka_ref, k_ref, v_ref, o_ref, km2_ref, qext_ref, acc_ref):
    t = pl.program_id(0)

    @pl.when(t < N0)
    def _scan_key_norms():
        ka = ka_ref[...]
        n2 = jnp.sum(ka * ka, axis=1, keepdims=True)
        bmax = jnp.max(n2, axis=0, keepdims=True)

        @pl.when(t == 0)
        def _():
            km2_ref[...] = bmax

        @pl.when(t > 0)
        def _():
            km2_ref[...] = jnp.maximum(km2_ref[...], bmax)

    @pl.when(t >= N0)
    def _readout():
        i = t - N0

        @pl.when(i == 0)
        def _():
            q = q_ref[...]
            qn = jnp.sum(q * q, axis=1, keepdims=True)
            u = jnp.sqrt(qn * km2_ref[...])
            qext_ref[:, 0:KD] = q * LOG2E
            qext_ref[:, KD : KD + 1] = -(u * LOG2E)
            acc_ref[...] = jnp.zeros_like(acc_ref)

        ones_k = jnp.ones((M_BLK, 1), jnp.float32)
        k_ext = jnp.concatenate([k_ref[...], ones_k], axis=1)
        s = jax.lax.dot_general(
            qext_ref[...],
            k_ext,
            (((1,), (1,)), ((), ())),
            preferred_element_type=jnp.float32,
        )
        p = jnp.exp2(s)
        v_ext = jnp.concatenate([v_ref[...], ones_k], axis=1)
        acc_ref[...] += jnp.dot(p, v_ext, preferred_element_type=jnp.float32)

        @pl.when(i == NB - 1)
        def _():
            o_ref[...] = acc_ref[:, 0:VD] / acc_ref[:, VD : VD + 1]


@jax.jit
def kernel(query, keys, values):
    return pl.pallas_call(
        _body,
        grid=(N0 + NB,),
        in_specs=[
            pl.BlockSpec((B, KD), lambda t: (0, 0)),
            pl.BlockSpec((M0_BLK, KD), lambda t: (jnp.minimum(t, N0 - 1), 0)),
            pl.BlockSpec((M_BLK, KD), lambda t: (jnp.maximum(t - N0, 0), 0)),
            pl.BlockSpec((M_BLK, VD), lambda t: (jnp.maximum(t - N0, 0), 0)),
        ],
        out_specs=pl.BlockSpec((B, VD), lambda t: (0, 0)),
        out_shape=jax.ShapeDtypeStruct((B, VD), jnp.float32),
        scratch_shapes=[
            pltpu.VMEM((1, 1), jnp.float32),
            pltpu.VMEM((B, KD + 1), jnp.float32),
            pltpu.VMEM((B, VD + 1), jnp.float32),
        ],
    )(query, keys, keys, values)
